# baseline (speedup 1.0000x reference)
import functools

import jax
import jax.numpy as jnp
from jax import lax
from jax.experimental import pallas as pl
from jax.experimental.pallas import tpu as pltpu
from jax.experimental.pallas import tpu_sc as plsc

L = 16


def _mf_kernel(B, D, num_cores, num_subcores):
    NW = num_cores * num_subcores
    bpw = B // NW
    mesh = plsc.VectorSubcoreMesh(
        core_axis_name="c", subcore_axis_name="s",
        num_cores=num_cores, num_subcores=num_subcores)

    @functools.partial(
        pl.kernel,
        out_type=jax.ShapeDtypeStruct((B,), jnp.float32),
        mesh=mesh,
        scratch_types=[
            pltpu.VMEM((bpw,), jnp.int32),
            pltpu.VMEM((bpw,), jnp.int32),
            pltpu.VMEM((bpw, D), jnp.float32),
            pltpu.VMEM((bpw, D), jnp.float32),
            pltpu.VMEM((bpw,), jnp.float32),
            pltpu.VMEM((bpw,), jnp.float32),
            pltpu.SemaphoreType.DMA,
        ],
        compiler_params=pltpu.CompilerParams(
            needs_layout_passes=False, use_tc_tiling_on_sc=False),
    )
    def mf(uid_hbm, iid_hbm, ut_hbm, it_hbm, bias_hbm, out_hbm,
           uidx_v, iidx_v, urows_v, irows_v, bias_v, out_v, sem):
        wid = lax.axis_index("s") * num_cores + lax.axis_index("c")
        base = wid * bpw

        pltpu.sync_copy(uid_hbm.at[pl.ds(base, bpw)], uidx_v)
        pltpu.sync_copy(iid_hbm.at[pl.ds(base, bpw)], iidx_v)
        cu = pltpu.async_copy(ut_hbm.at[uidx_v], urows_v, sem)
        ci = pltpu.async_copy(it_hbm.at[iidx_v], irows_v, sem)
        cb = pltpu.async_copy(bias_hbm.at[iidx_v], bias_v, sem)
        cu.wait()
        ci.wait()
        cb.wait()

        lane = lax.iota(jnp.int32, L)

        def group(g, carry):
            rvec = g * L + lane
            accs = [jnp.zeros((L,), jnp.float32) for _ in range(4)]
            for j in range(D):
                cvec = (lane + j) & (D - 1)
                u = plsc.load_gather(urows_v, [rvec, cvec])
                t = plsc.load_gather(irows_v, [rvec, cvec])
                accs[j % 4] = accs[j % 4] + u * t
            tot = (accs[0] + accs[1]) + (accs[2] + accs[3])
            tot = tot + bias_v[pl.ds(g * L, L)]
            out_v[pl.ds(g * L, L)] = tot
            return carry

        lax.fori_loop(0, bpw // L, group, 0)
        pltpu.sync_copy(out_v, out_hbm.at[pl.ds(base, bpw)])

    return mf


def kernel(user_ids, item_ids, user_table, item_table, item_bias_table):
    B = user_ids.shape[0]
    D = user_table.shape[1]
    bias_flat = item_bias_table.reshape((item_bias_table.shape[0],))
    mf = _mf_kernel(B, D, 2, 16)
    return mf(user_ids.astype(jnp.int32), item_ids.astype(jnp.int32),
              user_table, item_table, bias_flat)
